# baseline (speedup 1.0000x reference)
import jax
import jax.numpy as jnp
from jax.experimental import pallas as pl
from jax.experimental.pallas import tpu as pltpu


def _fma_body(idx_ref, site_ref, mask_ref, x_ref, o_ref):
    b = pl.program_id(0)
    i = idx_ref[b]
    o_ref[...] = x_ref[...] * mask_ref[i] + site_ref[i]


_ROWS = 512


def kernel(inputs, available_injection_sites, masks):
    B, H, W, C = inputs.shape
    n = available_injection_sites.shape[0]
    idx = jax.random.randint(jax.random.key(22), (B,), 0, n).astype(jnp.int32)
    sites = available_injection_sites.reshape(n)
    msk = masks.reshape(n)

    x = inputs.reshape(B, H * W, C)
    out = pl.pallas_call(
        _fma_body,
        grid_spec=pltpu.PrefetchScalarGridSpec(
            num_scalar_prefetch=3,
            grid=(B, (H * W) // _ROWS),
            in_specs=[
                pl.BlockSpec((1, _ROWS, C), lambda b, r, *_: (b, r, 0)),
            ],
            out_specs=pl.BlockSpec((1, _ROWS, C), lambda b, r, *_: (b, r, 0)),
        ),
        out_shape=jax.ShapeDtypeStruct((B, H * W, C), inputs.dtype),
        compiler_params=pltpu.CompilerParams(
            dimension_semantics=("parallel", "parallel"),
        ),
    )(idx, sites, msk, x)
    return out.reshape(B, H, W, C)


# speedup vs baseline: 1.2229x; 1.2229x over previous
import jax
import jax.numpy as jnp
from jax.experimental import pallas as pl
from jax.experimental.pallas import tpu as pltpu


_BB = 2


def _fma_body(idx_ref, site_ref, mask_ref, x_ref, o_ref):
    b = pl.program_id(0)
    for j in range(_BB):
        i = idx_ref[b * _BB + j]
        o_ref[j] = x_ref[j] * mask_ref[i] + site_ref[i]


def kernel(inputs, available_injection_sites, masks):
    B, H, W, C = inputs.shape
    n = available_injection_sites.shape[0]
    idx = jax.random.randint(jax.random.key(22), (B,), 0, n).astype(jnp.int32)
    sites = available_injection_sites.reshape(n)
    msk = masks.reshape(n)

    x = inputs.reshape(B, H * W, C)
    out = pl.pallas_call(
        _fma_body,
        grid_spec=pltpu.PrefetchScalarGridSpec(
            num_scalar_prefetch=3,
            grid=(B // _BB,),
            in_specs=[
                pl.BlockSpec((_BB, H * W, C), lambda b, *_: (b, 0, 0)),
            ],
            out_specs=pl.BlockSpec((_BB, H * W, C), lambda b, *_: (b, 0, 0)),
        ),
        out_shape=jax.ShapeDtypeStruct((B, H * W, C), inputs.dtype),
        compiler_params=pltpu.CompilerParams(
            dimension_semantics=("parallel",),
        ),
    )(idx, sites, msk, x)
    return out.reshape(B, H, W, C)


# speedup vs baseline: 1.2425x; 1.0160x over previous
import jax
import jax.numpy as jnp
from jax.experimental import pallas as pl
from jax.experimental.pallas import tpu as pltpu


_BB = 4


def _fma_body(idx_ref, site_ref, mask_ref, x_ref, o_ref):
    b = pl.program_id(0)
    for j in range(_BB):
        i = idx_ref[b * _BB + j]
        o_ref[j] = x_ref[j] * mask_ref[i] + site_ref[i]


def kernel(inputs, available_injection_sites, masks):
    B, H, W, C = inputs.shape
    n = available_injection_sites.shape[0]
    idx = jax.random.randint(jax.random.key(22), (B,), 0, n).astype(jnp.int32)
    sites = available_injection_sites.reshape(n)
    msk = masks.reshape(n)

    x = inputs.reshape(B, H * W, C)
    out = pl.pallas_call(
        _fma_body,
        grid_spec=pltpu.PrefetchScalarGridSpec(
            num_scalar_prefetch=3,
            grid=(B // _BB,),
            in_specs=[
                pl.BlockSpec((_BB, H * W, C), lambda b, *_: (b, 0, 0)),
            ],
            out_specs=pl.BlockSpec((_BB, H * W, C), lambda b, *_: (b, 0, 0)),
        ),
        out_shape=jax.ShapeDtypeStruct((B, H * W, C), inputs.dtype),
        compiler_params=pltpu.CompilerParams(
            dimension_semantics=("parallel",),
        ),
    )(idx, sites, msk, x)
    return out.reshape(B, H, W, C)
